# baseline (speedup 1.0000x reference)
import functools

import jax
import jax.numpy as jnp
from jax import lax
from jax.experimental import pallas as pl
from jax.experimental.pallas import tpu as pltpu
from jax.experimental.pallas import tpu_sc as plsc

_D = 64
_CH = 512


@functools.lru_cache(maxsize=None)
def _make(B: int):
    info = plsc.get_sparse_core_info()
    nw = info.num_cores * info.num_subcores
    b_per_w = B // nw
    n_chunks = b_per_w // _CH
    assert b_per_w * nw == B and n_chunks * _CH == b_per_w

    mesh = plsc.VectorSubcoreMesh(core_axis_name="c", subcore_axis_name="s")

    @functools.partial(
        pl.kernel,
        mesh=mesh,
        out_type=jax.ShapeDtypeStruct((B, _D), jnp.float32),
        scratch_types=[
            pltpu.VMEM((_CH,), jnp.int32),
            pltpu.VMEM((_CH, _D), jnp.float32),
            pltpu.SemaphoreType.DMA,
        ],
        compiler_params=pltpu.CompilerParams(use_tc_tiling_on_sc=False),
    )
    def k(xs_hbm, table_hbm, out_hbm, idx_v, rows_v, sem):
        wid = lax.axis_index("s") * info.num_cores + lax.axis_index("c")
        base = wid * b_per_w

        def body(c, carry):
            off = base + c * _CH
            pltpu.sync_copy(xs_hbm.at[pl.ds(off, _CH)], idx_v)
            pltpu.async_copy(table_hbm.at[idx_v], rows_v, sem).wait()
            pltpu.sync_copy(rows_v, out_hbm.at[pl.ds(off, _CH)])
            return carry

        lax.fori_loop(0, n_chunks, body, 0)

    return k


def kernel(xs, table):
    b, h = xs.shape
    flat = xs.reshape(b * h)
    out = _make(b * h)(flat, table)
    return out.reshape(b, h, _D)


# speedup vs baseline: 1.0379x; 1.0379x over previous
import functools

import jax
import jax.numpy as jnp
from jax import lax
from jax.experimental import pallas as pl
from jax.experimental.pallas import tpu as pltpu
from jax.experimental.pallas import tpu_sc as plsc

_D = 64
_CH = 640


@functools.lru_cache(maxsize=None)
def _make(B: int):
    info = plsc.get_sparse_core_info()
    nw = info.num_cores * info.num_subcores
    b_per_w = B // nw
    n = b_per_w // _CH
    assert b_per_w * nw == B and n * _CH == b_per_w and n >= 4 and n % 2 == 0

    mesh = plsc.VectorSubcoreMesh(core_axis_name="c", subcore_axis_name="s")

    @functools.partial(
        pl.kernel,
        mesh=mesh,
        out_type=jax.ShapeDtypeStruct((B, _D), jnp.float32),
        scratch_types=[
            pltpu.VMEM((_CH,), jnp.int32),
            pltpu.VMEM((_CH,), jnp.int32),
            pltpu.VMEM((_CH, _D), jnp.float32),
            pltpu.VMEM((_CH, _D), jnp.float32),
            pltpu.SemaphoreType.DMA,
            pltpu.SemaphoreType.DMA,
            pltpu.SemaphoreType.DMA,
            pltpu.SemaphoreType.DMA,
            pltpu.SemaphoreType.DMA,
            pltpu.SemaphoreType.DMA,
        ],
        compiler_params=pltpu.CompilerParams(use_tc_tiling_on_sc=False),
    )
    def k(xs_hbm, table_hbm, out_hbm,
          idx0, idx1, rows0, rows1, si0, si1, sg0, sg1, sw0, sw1):
        wid = lax.axis_index("s") * info.num_cores + lax.axis_index("c")
        base = wid * b_per_w
        idx, rows = (idx0, idx1), (rows0, rows1)
        si, sg, sw = (si0, si1), (sg0, sg1), (sw0, sw1)

        def i_start(c, b):
            pltpu.async_copy(xs_hbm.at[pl.ds(base + c * _CH, _CH)], idx[b], si[b])

        def i_wait(b):
            pltpu.make_async_copy(
                xs_hbm.at[pl.ds(base, _CH)], idx[b], si[b]).wait()

        def g_start(b):
            pltpu.async_copy(table_hbm.at[idx[b]], rows[b], sg[b])

        def g_wait(b):
            pltpu.make_async_copy(table_hbm.at[idx[b]], rows[b], sg[b]).wait()

        def w_start(c, b):
            pltpu.async_copy(rows[b], out_hbm.at[pl.ds(base + c * _CH, _CH)], sw[b])

        def w_wait(b):
            pltpu.make_async_copy(
                rows[b], out_hbm.at[pl.ds(base, _CH)], sw[b]).wait()

        i_start(0, 0)
        i_start(1, 1)
        i_wait(0)
        g_start(0)
        g_wait(0)
        w_start(0, 0)
        i_start(2, 0)
        i_wait(1)
        g_start(1)

        def body(g2, carry):
            c = 2 + 2 * g2
            for b in (0, 1):
                cc = c + b
                nb = 1 - b
                g_wait(nb)
                w_start(cc - 1, nb)
                i_start(lax.rem(cc + 1, n), nb)
                w_wait(b)
                i_wait(b)
                g_start(b)
            return carry

        lax.fori_loop(0, (n - 2) // 2, body, 0)

        g_wait(1)
        w_start(n - 1, 1)
        w_wait(0)
        w_wait(1)
        i_wait(0)

    return k


def kernel(xs, table):
    b, h = xs.shape
    flat = xs.reshape(b * h)
    out = _make(b * h)(flat, table)
    return out.reshape(b, h, _D)
